# Initial kernel scaffold; baseline (speedup 1.0000x reference)
import jax
import jax.numpy as jnp
from jax.experimental import pallas as pl


def kernel(vfeat, efeat, DV2, invDE, edge_index, W_v, b_v, W_e, b_e):
    raise NotImplementedError("write your pallas kernel here")


# speedup vs baseline: 10.7254x; 10.7254x over previous
import jax
import jax.numpy as jnp
from jax import lax
from jax.experimental import pallas as pl
from jax.experimental.pallas import tpu as pltpu
from jax.experimental.pallas import tpu_sc as plsc

N = 10000
E = 320000
D = 128
D_E = 16
NC, NS = 2, 16
NW = NC * NS
EPW = E // NW
CH = 80
NCHUNK = EPW // CH
RPT = 624
RPT_LAST = N - RPT * (NS - 1)
RB = 2000


def _prep_body(vfeat_ref, w_ref, b_ref, dv2_ref, out_ref):
    wh = jnp.dot(vfeat_ref[...], w_ref[...], preferred_element_type=jnp.float32)
    out_ref[...] = (wh + b_ref[...]) * dv2_ref[...]


_prep = pl.pallas_call(
    _prep_body,
    grid=(N // RB,),
    in_specs=[
        pl.BlockSpec((RB, D), lambda i: (i, 0)),
        pl.BlockSpec((D, D), lambda i: (0, 0)),
        pl.BlockSpec((1, D), lambda i: (0, 0)),
        pl.BlockSpec((RB, 1), lambda i: (i, 0)),
    ],
    out_specs=pl.BlockSpec((RB, D), lambda i: (i, 0)),
    out_shape=jax.ShapeDtypeStruct((N, D), jnp.float32),
)


def _mid_body(p_ref, inv_ref, we_ref, be_ref, e2_ref, eout_ref):
    en = p_ref[0] + p_ref[1]
    e2_ref[...] = en * inv_ref[...]
    eout_ref[...] = (
        jnp.dot(en, we_ref[...], preferred_element_type=jnp.float32) + be_ref[...]
    )


_mid = pl.pallas_call(
    _mid_body,
    grid=(N // RB,),
    in_specs=[
        pl.BlockSpec((2, RB, D), lambda i: (0, i, 0)),
        pl.BlockSpec((RB, 1), lambda i: (i, 0)),
        pl.BlockSpec((D, D_E), lambda i: (0, 0)),
        pl.BlockSpec((1, D_E), lambda i: (0, 0)),
    ],
    out_specs=[
        pl.BlockSpec((RB, D), lambda i: (i, 0)),
        pl.BlockSpec((RB, D_E), lambda i: (i, 0)),
    ],
    out_shape=[
        jax.ShapeDtypeStruct((N, D), jnp.float32),
        jax.ShapeDtypeStruct((N, D_E), jnp.float32),
    ],
)


def _final_body(s_ref, dv2_ref, out_ref):
    out_ref[...] = jnp.maximum((s_ref[0] + s_ref[1]) * dv2_ref[...], 0.0)


_final = pl.pallas_call(
    _final_body,
    grid=(N // RB,),
    in_specs=[
        pl.BlockSpec((2, RB, D), lambda i: (0, i, 0)),
        pl.BlockSpec((RB, 1), lambda i: (i, 0)),
    ],
    out_specs=pl.BlockSpec((RB, D), lambda i: (i, 0)),
    out_shape=jax.ShapeDtypeStruct((N, D), jnp.float32),
)


def _sc_body(table, gidx, sidx, zeros, out, idx_g, idx_s, rows, acc, sem):
    cid = lax.axis_index("c")
    sid = lax.axis_index("s")
    w = sid * NC + cid
    start = pl.multiple_of(sid * RPT, 8)
    @pl.when(sid < NS - 1)
    def _():
        pltpu.sync_copy(zeros.at[pl.ds(0, RPT)], acc.at[pl.ds(start, RPT), :])

    @pl.when(sid == NS - 1)
    def _():
        pltpu.sync_copy(zeros, acc.at[pl.ds(start, RPT_LAST), :])

    plsc.subcore_barrier()

    def step(i, carry):
        off = pl.multiple_of(w * EPW + i * CH, 8)
        pltpu.sync_copy(gidx.at[pl.ds(off, CH)], idx_g)
        pltpu.sync_copy(sidx.at[pl.ds(off, CH)], idx_s)
        pltpu.async_copy(table.at[idx_g], rows, sem).wait()
        pltpu.sync_copy(rows, acc.at[idx_s], add=True)
        return carry

    lax.fori_loop(0, NCHUNK, step, 0)
    plsc.subcore_barrier()

    @pl.when(sid < NS - 1)
    def _():
        pltpu.sync_copy(
            acc.at[pl.ds(start, RPT), :], out.at[cid, pl.ds(start, RPT), :]
        )

    @pl.when(sid == NS - 1)
    def _():
        pltpu.sync_copy(
            acc.at[pl.ds(start, RPT_LAST), :],
            out.at[cid, pl.ds(start, RPT_LAST), :],
        )


_sc_scatter = pl.kernel(
    _sc_body,
    out_type=jax.ShapeDtypeStruct((NC, N, D), jnp.float32),
    mesh=plsc.VectorSubcoreMesh(
        core_axis_name="c", subcore_axis_name="s", num_cores=NC, num_subcores=NS
    ),
    scratch_types=[
        pltpu.VMEM((CH,), jnp.int32),
        pltpu.VMEM((CH,), jnp.int32),
        pltpu.VMEM((CH, D), jnp.float32),
        pltpu.VMEM_SHARED((N, D), jnp.float32),
        pltpu.SemaphoreType.DMA,
    ],
)


def kernel(vfeat, efeat, DV2, invDE, edge_index, W_v, b_v, W_e, b_e):
    src = edge_index[0]
    dst = edge_index[1]
    zeros = jnp.zeros((RPT_LAST, D), jnp.float32)
    dv2c = DV2.reshape(N, 1)
    t1 = _prep(vfeat, W_v, b_v.reshape(1, D), dv2c)
    p = _sc_scatter(t1, src, dst, zeros)
    e2, efeat_out = _mid(p, invDE.reshape(N, 1), W_e, b_e.reshape(1, D_E))
    s = _sc_scatter(e2, dst, src, zeros)
    vfeat_out = _final(s, dv2c)
    return (vfeat_out, efeat_out)
